# baseline (speedup 1.0000x reference)
import functools

import jax
import jax.numpy as jnp
from jax import lax
from jax.experimental import pallas as pl
from jax.experimental.pallas import tpu as pltpu
from jax.experimental.pallas import tpu_sc as plsc

ND = 50000
NP_ = 50176
CHUNK = 12544
CT = CHUNK + 8
TPB = CHUNK // 16
EB = 128
EBA = 96
BM = 256
GPAD = 100352
GB = GPAD // BM


@functools.lru_cache(maxsize=None)
def _bin_call(epad):
    epw = epad // 32
    nsteps = epw // EB
    cap = epw + EB
    mesh = plsc.VectorSubcoreMesh(core_axis_name="c", subcore_axis_name="s")

    @functools.partial(
        pl.kernel,
        mesh=mesh,
        out_type=[
            jax.ShapeDtypeStruct((32 * 4 * cap,), jnp.int32),
            jax.ShapeDtypeStruct((32 * 4 * cap,), jnp.int32),
            jax.ShapeDtypeStruct((32, 16), jnp.int32),
        ],
        scratch_types=[
            pltpu.VMEM((EB,), jnp.int32), pltpu.VMEM((EB,), jnp.int32),
            pltpu.VMEM((EB,), jnp.int32), pltpu.VMEM((EB,), jnp.int32),
            pltpu.VMEM((EB,), jnp.int32), pltpu.VMEM((EB,), jnp.int32),
            pltpu.VMEM((1, 16), jnp.int32),
            pltpu.SemaphoreType.DMA, pltpu.SemaphoreType.DMA,
        ],
    )
    def binker(src_h, dst_h, bsrc_h, bdl_h, cnt_h,
               si, di, pos_v, dlb_v, zpad_v, tpad_v, cnt_v, s1, s2):
        core = lax.axis_index("c")
        sub = lax.axis_index("s")
        w = sub * 2 + core
        ebase = w * epw
        wbase = w * (4 * cap)
        io = lax.iota(jnp.int32, 16)
        zero16 = jnp.zeros((16,), jnp.int32)
        trash16 = jnp.full((16,), CHUNK, jnp.int32)
        for j in range(8):
            zpad_v[pl.ds(j * 16, 16)] = zero16
            tpad_v[pl.ds(j * 16, 16)] = trash16

        def prefix(mi):
            inc = mi
            for kk in (1, 2, 4, 8):
                sh = jnp.where(io >= kk, inc[jnp.maximum(io - kk, 0)], 0)
                inc = inc + sh
            return inc

        def step(k, cnts):
            off = ebase + k * EB
            pltpu.sync_copy(src_h.at[pl.ds(off, EB)], si)
            pltpu.sync_copy(dst_h.at[pl.ds(off, EB)], di)
            for j in range(8):
                d16 = di[pl.ds(j * 16, 16)]
                base = jnp.where(
                    d16 >= 3 * CHUNK, 3 * CHUNK,
                    jnp.where(d16 >= 2 * CHUNK, 2 * CHUNK,
                              jnp.where(d16 >= CHUNK, CHUNK, 0)))
                dl16 = d16 - base
                gp = zero16
                new_cnts = []
                for b in range(4):
                    m = base == b * CHUNK
                    mi = jnp.where(m, 1, 0)
                    inc = prefix(mi)
                    gp = gp + jnp.where(
                        m, (b * cap + cnts[b]) + inc - mi, 0)
                    new_cnts.append(cnts[b] + inc[15])
                cnts = tuple(new_cnts)
                pos_v[pl.ds(j * 16, 16)] = wbase + gp
                dlb_v[pl.ds(j * 16, 16)] = dl16
            pltpu.async_copy(si, bsrc_h.at[pos_v], s1)
            pltpu.async_copy(dlb_v, bdl_h.at[pos_v], s2)
            pltpu.make_async_copy(si, bsrc_h.at[pos_v], s1).wait()
            pltpu.make_async_copy(dlb_v, bdl_h.at[pos_v], s2).wait()
            return cnts

        z = jnp.int32(0)
        cnts = lax.fori_loop(0, nsteps, step, (z, z, z, z))
        for b in range(4):
            for j in range(8):
                pos_v[pl.ds(j * 16, 16)] = (
                    (wbase + b * cap + cnts[b]) + j * 16 + io)
            pltpu.async_copy(zpad_v, bsrc_h.at[pos_v], s1)
            pltpu.async_copy(tpad_v, bdl_h.at[pos_v], s2)
            pltpu.make_async_copy(zpad_v, bsrc_h.at[pos_v], s1).wait()
            pltpu.make_async_copy(tpad_v, bdl_h.at[pos_v], s2).wait()
        cv = jnp.where(io == 0, cnts[0],
                       jnp.where(io == 1, cnts[1],
                                 jnp.where(io == 2, cnts[2],
                                           jnp.where(io == 3, cnts[3], 0))))
        cnt_v[0, pl.ds(0, 16)] = cv
        pltpu.sync_copy(cnt_v, cnt_h.at[pl.ds(w, 1)])

    return binker


@functools.lru_cache(maxsize=None)
def _agg_call(epad):
    epw = epad // 32
    cap = epw + EB
    mesh = plsc.VectorSubcoreMesh(core_axis_name="c", subcore_axis_name="s")

    @functools.partial(
        pl.kernel,
        mesh=mesh,
        out_type=jax.ShapeDtypeStruct((NP_, 128), jnp.float32),
        scratch_types=[
            pltpu.VMEM((EB,), jnp.int32), pltpu.VMEM((EB,), jnp.int32),
            pltpu.VMEM((EB, 128), jnp.float32),
            pltpu.VMEM((64, 128), jnp.float32),
            pltpu.VMEM((32, 16), jnp.int32),
            pltpu.VMEM_SHARED((CT, 128), jnp.float32),
            pltpu.SemaphoreType.DMA,
        ],
    )
    def agg(bsrc_h, bdl_h, cnt_h, y_h, out_h,
            si, di, rows, zb_v, cnt_v, agg_s, sem):
        core = lax.axis_index("c")
        sub = lax.axis_index("s")
        zero16 = jnp.zeros((16,), jnp.float32)

        def zrow(i, c):
            for j in range(8):
                zb_v[i, pl.ds(j * 16, 16)] = zero16
            return c

        lax.fori_loop(0, 64, zrow, 0)
        pltpu.sync_copy(cnt_h, cnt_v)

        for p in range(2):
            b = 2 * p + core
            lo = b * CHUNK
            for t in range(12):
                pltpu.sync_copy(zb_v.at[pl.ds(0, 64)],
                                agg_s.at[pl.ds(sub * TPB + t * 64, 64)])
            pltpu.sync_copy(zb_v.at[pl.ds(0, 16)],
                            agg_s.at[pl.ds(sub * TPB + 768, 16)])
            plsc.subcore_barrier()

            for w in (2 * sub, 2 * sub + 1):
                vw = cnt_v[w, pl.ds(0, 16)]
                cb = jnp.where(core == 0, vw[2 * p], vw[2 * p + 1])
                ngroups = (cb + EB - 1) >> 7
                rbase = (w * 4 + b) * cap

                def grp(g, c):
                    off = rbase + g * EB
                    pltpu.sync_copy(bsrc_h.at[pl.ds(off, EB)], si)
                    pltpu.sync_copy(bdl_h.at[pl.ds(off, EB)], di)
                    pltpu.async_copy(y_h.at[si], rows, sem).wait()
                    pltpu.sync_copy(rows, agg_s.at[di], add=True)
                    return c

                lax.fori_loop(0, ngroups, grp, 0)

            plsc.subcore_barrier()
            pltpu.sync_copy(
                agg_s.at[pl.ds(sub * TPB, TPB)],
                out_h.at[pl.ds(lo + sub * TPB, TPB)],
            )
            plsc.subcore_barrier()

    return agg


@functools.lru_cache(maxsize=None)
def _gather_call(n):
    per_w = n // 32
    nsteps = per_w // EB
    mesh = plsc.VectorSubcoreMesh(core_axis_name="c", subcore_axis_name="s")

    npairs = nsteps // 2
    assert nsteps % 2 == 0

    @functools.partial(
        pl.kernel,
        mesh=mesh,
        out_type=jax.ShapeDtypeStruct((n, 128), jnp.float32),
        scratch_types=[
            pltpu.VMEM((EB,), jnp.int32), pltpu.VMEM((EB,), jnp.int32),
            pltpu.VMEM((EB, 128), jnp.float32),
            pltpu.VMEM((EB, 128), jnp.float32),
            pltpu.SemaphoreType.DMA, pltpu.SemaphoreType.DMA,
            pltpu.SemaphoreType.DMA, pltpu.SemaphoreType.DMA,
            pltpu.SemaphoreType.DMA, pltpu.SemaphoreType.DMA,
        ],
    )
    def gat(idx_h, z_h, out_h, i0, i1, rows0, rows1,
            si0, si1, sg0, sg1, so0, so1):
        core = lax.axis_index("c")
        sub = lax.axis_index("s")
        base = (sub * 2 + core) * per_w
        pltpu.async_copy(idx_h.at[pl.ds(base, EB)], i0, si0)
        pltpu.make_async_copy(idx_h.at[pl.ds(base, EB)], i0, si0).wait()
        pltpu.async_copy(z_h.at[i0], rows0, sg0)

        def pair(k, first):
            off0 = base + 2 * k * EB
            off1 = off0 + EB
            off2 = off0 + 2 * EB
            pltpu.async_copy(idx_h.at[pl.ds(off1, EB)], i1, si1)
            pltpu.make_async_copy(z_h.at[i0], rows0, sg0).wait()
            pltpu.async_copy(rows0, out_h.at[pl.ds(off0, EB)], so0)
            pltpu.make_async_copy(idx_h.at[pl.ds(off1, EB)], i1, si1).wait()
            if not first:
                pltpu.make_async_copy(rows1, out_h.at[pl.ds(off1, EB)],
                                      so1).wait()
            pltpu.async_copy(z_h.at[i1], rows1, sg1)
            pltpu.async_copy(idx_h.at[pl.ds(off2, EB)], i0, si0)
            pltpu.make_async_copy(z_h.at[i1], rows1, sg1).wait()
            pltpu.async_copy(rows1, out_h.at[pl.ds(off1, EB)], so1)
            pltpu.make_async_copy(idx_h.at[pl.ds(off2, EB)], i0, si0).wait()
            pltpu.make_async_copy(rows0, out_h.at[pl.ds(off0, EB)], so0).wait()
            pltpu.async_copy(z_h.at[i0], rows0, sg0)

        pair(0, True)
        lax.fori_loop(1, npairs, lambda k, c: (pair(k, False), c)[1], 0)
        pltpu.make_async_copy(z_h.at[i0], rows0, sg0).wait()
        pltpu.make_async_copy(rows1, out_h.at[pl.ds(base, EB)], so1).wait()

    return gat


@functools.lru_cache(maxsize=None)
def _mm_call(k):
    def body(x_ref, w_ref, b_ref, *o_refs):
        x = x_ref[...]
        for t in range(k):
            o_refs[t][...] = (
                jnp.dot(x, w_ref[:, t * 128:(t + 1) * 128],
                        preferred_element_type=jnp.float32)
                + b_ref[0, t * 128:(t + 1) * 128][None, :]
            )

    return pl.pallas_call(
        body,
        grid=(NP_ // BM,),
        in_specs=[
            pl.BlockSpec((BM, 128), lambda i: (i, 0)),
            pl.BlockSpec((128, 128 * k), lambda i: (0, 0)),
            pl.BlockSpec((8, 128 * k), lambda i: (0, 0)),
        ],
        out_specs=[pl.BlockSpec((BM, 128), lambda i: (i, 0))] * k,
        out_shape=[jax.ShapeDtypeStruct((NP_, 128), jnp.float32)] * k,
    )


@functools.lru_cache(maxsize=None)
def _comb_call(nt, relu):
    def body(*refs):
        o_ref = refs[-1]
        acc = None
        for t in range(nt):
            a = refs[2 * t][...] + refs[2 * t + 1][...]
            n2 = jnp.sum(a * a, axis=1, keepdims=True)
            a = a / jnp.maximum(jnp.sqrt(n2), 1e-12)
            acc = a if acc is None else acc + a
        if relu:
            acc = jnp.maximum(acc, 0.0)
        o_ref[...] = acc

    return pl.pallas_call(
        body,
        grid=(NP_ // BM,),
        in_specs=[pl.BlockSpec((BM, 128), lambda i: (i, 0))] * (2 * nt),
        out_specs=pl.BlockSpec((BM, 128), lambda i: (i, 0)),
        out_shape=jax.ShapeDtypeStruct((NP_, 128), jnp.float32),
    )


@functools.lru_cache(maxsize=None)
def _dec_call(blk_a, blk_b):
    DB = 8 * BM

    def body(za_ref, zb_ref, r_ref, o_ref):
        t = jnp.dot(za_ref[...], r_ref[...],
                    preferred_element_type=jnp.float32) * zb_ref[...]
        o_ref[...] = jnp.sum(t, axis=1).reshape(8, BM)

    return pl.pallas_call(
        body,
        grid=(GPAD // DB,),
        in_specs=[
            pl.BlockSpec((DB, 128), lambda i, o=blk_a: (i + o, 0)),
            pl.BlockSpec((DB, 128), lambda i, o=blk_b: (i + o, 0)),
            pl.BlockSpec((128, 128), lambda i: (0, 0)),
        ],
        out_specs=pl.BlockSpec((8, BM), lambda i: (i, 0)),
        out_shape=jax.ShapeDtypeStruct((GB, BM), jnp.float32),
    )


def _pad_rows(x):
    return jnp.concatenate(
        [x, jnp.zeros((NP_ - x.shape[0], x.shape[1]), x.dtype)], axis=0)


def _pad_edges(ei, epad):
    e = ei.shape[1]
    src = jnp.concatenate(
        [ei[0].astype(jnp.int32), jnp.zeros((epad - e,), jnp.int32)])
    dst = jnp.concatenate(
        [ei[1].astype(jnp.int32), jnp.full((epad - e,), ND, jnp.int32)])
    return src, dst


def _pad_idx(ix, extra=0):
    return jnp.concatenate(
        [ix.astype(jnp.int32),
         jnp.zeros((GPAD + extra - ix.shape[0],), jnp.int32)])


def kernel(x_drug, x_gene, ei_gene_interact_gene, ei_drug_has_target_gene, ei_gene_get_target_drug, ei_drug_rel0_drug, ei_drug_rel1_drug, Wm_gene_interact_gene_0, bm_gene_interact_gene_0, Ws_gene_interact_gene_0, bs_gene_interact_gene_0, Wm_drug_has_target_gene_0, bm_drug_has_target_gene_0, Ws_drug_has_target_gene_0, bs_drug_has_target_gene_0, Wm_gene_get_target_drug_0, bm_gene_get_target_drug_0, Ws_gene_get_target_drug_0, bs_gene_get_target_drug_0, Wm_drug_rel0_drug_0, bm_drug_rel0_drug_0, Ws_drug_rel0_drug_0, bs_drug_rel0_drug_0, Wm_drug_rel1_drug_0, bm_drug_rel1_drug_0, Ws_drug_rel1_drug_0, bs_drug_rel1_drug_0, Wm_gene_interact_gene_1, bm_gene_interact_gene_1, Ws_gene_interact_gene_1, bs_gene_interact_gene_1, Wm_drug_has_target_gene_1, bm_drug_has_target_gene_1, Ws_drug_has_target_gene_1, bs_drug_has_target_gene_1, Wm_gene_get_target_drug_1, bm_gene_get_target_drug_1, Ws_gene_get_target_drug_1, bs_gene_get_target_drug_1, Wm_drug_rel0_drug_1, bm_drug_rel0_drug_1, Ws_drug_rel0_drug_1, bs_drug_rel0_drug_1, Wm_drug_rel1_drug_1, bm_drug_rel1_drug_1, Ws_drug_rel1_drug_1, bs_drug_rel1_drug_1, R_bilinear_rel0, R_dedicom, D_dedicom_rel1):
    Wl = [
        (Wm_gene_interact_gene_0, bm_gene_interact_gene_0,
         Ws_gene_interact_gene_0, bs_gene_interact_gene_0,
         Wm_drug_has_target_gene_0, bm_drug_has_target_gene_0,
         Ws_drug_has_target_gene_0, bs_drug_has_target_gene_0,
         Wm_gene_get_target_drug_0, bm_gene_get_target_drug_0,
         Ws_gene_get_target_drug_0, bs_gene_get_target_drug_0,
         Wm_drug_rel0_drug_0, bm_drug_rel0_drug_0,
         Ws_drug_rel0_drug_0, bs_drug_rel0_drug_0,
         Wm_drug_rel1_drug_0, bm_drug_rel1_drug_0,
         Ws_drug_rel1_drug_0, bs_drug_rel1_drug_0),
        (Wm_gene_interact_gene_1, bm_gene_interact_gene_1,
         Ws_gene_interact_gene_1, bs_gene_interact_gene_1,
         Wm_drug_has_target_gene_1, bm_drug_has_target_gene_1,
         Ws_drug_has_target_gene_1, bs_drug_has_target_gene_1,
         Wm_gene_get_target_drug_1, bm_gene_get_target_drug_1,
         Ws_gene_get_target_drug_1, bs_gene_get_target_drug_1,
         Wm_drug_rel0_drug_1, bm_drug_rel0_drug_1,
         Ws_drug_rel0_drug_1, bs_drug_rel0_drug_1,
         Wm_drug_rel1_drug_1, bm_drug_rel1_drug_1,
         Ws_drug_rel1_drug_1, bs_drug_rel1_drug_1),
    ]
    src_gg, dst_gg = _pad_edges(ei_gene_interact_gene, 204800)
    src_dg, dst_dg = _pad_edges(ei_drug_has_target_gene, 102400)
    src_gd, dst_gd = _pad_edges(ei_gene_get_target_drug, 102400)
    src_d0, dst_d0 = _pad_edges(ei_drug_rel0_drug, 102400)
    src_d1, dst_d1 = _pad_edges(ei_drug_rel1_drug, 102400)
    bin_gg = _bin_call(204800)(src_gg, dst_gg)
    bin_dg = _bin_call(102400)(src_dg, dst_dg)
    bin_gd = _bin_call(102400)(src_gd, dst_gd)
    bin_d0 = _bin_call(102400)(src_d0, dst_d0)
    bin_d1 = _bin_call(102400)(src_d1, dst_d1)

    xg = _pad_rows(x_gene)
    xd = _pad_rows(x_drug)
    for l in range(2):
        (Wm_gg, bm_gg, Ws_gg, bs_gg,
         Wm_dg, bm_dg, Ws_dg, bs_dg,
         Wm_gd, bm_gd, Ws_gd, bs_gd,
         Wm_d0, bm_d0, Ws_d0, bs_d0,
         Wm_d1, bm_d1, Ws_d1, bs_d1) = Wl[l]
        Wg = jnp.concatenate([Wm_gg, Wm_gd, Ws_gg, Ws_dg], axis=1)
        bg = jnp.tile(jnp.concatenate([bm_gg, bm_gd, bs_gg, bs_dg])[None, :],
                      (8, 1))
        Wd = jnp.concatenate([Wm_dg, Wm_d0, Wm_d1, Ws_gd, Ws_d0, Ws_d1],
                             axis=1)
        bd = jnp.tile(
            jnp.concatenate([bm_dg, bm_d0, bm_d1, bs_gd, bs_d0, bs_d1])[None, :],
            (8, 1))
        Ymgg, Ymgd, Sgg, Sdg = _mm_call(4)(xg, Wg, bg)
        Ymdg, Ymd0, Ymd1, Sgd, Sd0, Sd1 = _mm_call(6)(xd, Wd, bd)
        agg_gg = _agg_call(204800)(*bin_gg, Ymgg)
        agg_dg = _agg_call(102400)(*bin_dg, Ymdg)
        agg_gd = _agg_call(102400)(*bin_gd, Ymgd)
        agg_d0 = _agg_call(102400)(*bin_d0, Ymd0)
        agg_d1 = _agg_call(102400)(*bin_d1, Ymd1)
        relu = l == 0
        xg = _comb_call(2, relu)(agg_gg, Sgg, agg_dg, Sdg)
        xd = _comb_call(3, relu)(agg_gd, Sgd, agg_d0, Sd0, agg_d1, Sd1)

    idx_cat = jnp.concatenate([
        _pad_idx(ei_drug_rel0_drug[0]), _pad_idx(ei_drug_rel0_drug[1]),
        _pad_idx(ei_drug_rel1_drug[0]),
        _pad_idx(ei_drug_rel1_drug[1], extra=EB)])
    Zr = _gather_call(4 * GPAD)(idx_cat, xd)
    R1 = (D_dedicom_rel1[:, None] * R_dedicom) * D_dedicom_rel1[None, :]
    s0 = _dec_call(0, 49)(Zr, Zr, R_bilinear_rel0)
    s1 = _dec_call(98, 147)(Zr, Zr, R1)
    return jnp.concatenate(
        [s0.reshape(-1)[:100000], s1.reshape(-1)[:100000]])


# speedup vs baseline: 2.3762x; 2.3762x over previous
import functools

import jax
import jax.numpy as jnp
from jax import lax
from jax.experimental import pallas as pl
from jax.experimental.pallas import tpu as pltpu
from jax.experimental.pallas import tpu_sc as plsc

ND = 50000
NP_ = 50176
CHUNK = 12544
CT = CHUNK + 8
TPB = CHUNK // 16
EB = 128
EBA = 96
BM = 256
GPAD = 100352
GB = GPAD // BM


@functools.lru_cache(maxsize=None)
def _agg_call(epad):
    epw = epad // 16
    nsteps = epw // EB
    mesh = plsc.VectorSubcoreMesh(core_axis_name="c", subcore_axis_name="s")

    @functools.partial(
        pl.kernel,
        mesh=mesh,
        out_type=jax.ShapeDtypeStruct((NP_, 128), jnp.float32),
        scratch_types=[
            pltpu.VMEM((2 * EB,), jnp.int32),
            pltpu.VMEM((EB,), jnp.int32),
            pltpu.VMEM((EB, 128), jnp.float32),
            pltpu.VMEM((16, 128), jnp.float32),
            pltpu.VMEM_SHARED((CT, 128), jnp.float32),
            pltpu.SemaphoreType.DMA,
        ],
    )
    def agg(sd_h, y_h, out_h, sd_v, dl_v, rows_v, zb_v, agg_s, sem):
        core = lax.axis_index("c")
        sub = lax.axis_index("s")
        ebase = sub * epw * 2
        zero16 = jnp.zeros((16,), jnp.float32)
        for r in range(16):
            for j in range(8):
                zb_v[r, pl.ds(j * 16, 16)] = zero16
        for p in range(2):
            lo = (p * 2 + core) * CHUNK
            for t in range(TPB // 16):
                pltpu.sync_copy(zb_v, agg_s.at[pl.ds(sub * TPB + t * 16, 16)])
            plsc.subcore_barrier()

            def step(i, carry):
                off = ebase + i * (2 * EB)
                pltpu.sync_copy(sd_h.at[pl.ds(off, 2 * EB)], sd_v)
                pltpu.async_copy(y_h.at[sd_v.at[pl.ds(0, EB)]], rows_v,
                                 sem).wait()
                for j in range(8):
                    d16 = sd_v[pl.ds(EB + j * 16, 16)]
                    m = (d16 >= lo) & (d16 < lo + CHUNK)
                    dl_v[pl.ds(j * 16, 16)] = jnp.where(m, d16 - lo, CHUNK)
                pltpu.sync_copy(rows_v, agg_s.at[dl_v], add=True)
                return carry

            lax.fori_loop(0, nsteps, step, 0)
            plsc.subcore_barrier()
            pltpu.sync_copy(
                agg_s.at[pl.ds(sub * TPB, TPB)],
                out_h.at[pl.ds(lo + sub * TPB, TPB)],
            )
            plsc.subcore_barrier()

    return agg


@functools.lru_cache(maxsize=None)
def _gather_call(n):
    per_w = n // 32
    nsteps = per_w // EB
    mesh = plsc.VectorSubcoreMesh(core_axis_name="c", subcore_axis_name="s")

    npairs = nsteps // 2
    assert nsteps % 2 == 0

    @functools.partial(
        pl.kernel,
        mesh=mesh,
        out_type=jax.ShapeDtypeStruct((n, 128), jnp.float32),
        scratch_types=[
            pltpu.VMEM((EB,), jnp.int32), pltpu.VMEM((EB,), jnp.int32),
            pltpu.VMEM((EB, 128), jnp.float32),
            pltpu.VMEM((EB, 128), jnp.float32),
            pltpu.SemaphoreType.DMA, pltpu.SemaphoreType.DMA,
            pltpu.SemaphoreType.DMA, pltpu.SemaphoreType.DMA,
            pltpu.SemaphoreType.DMA, pltpu.SemaphoreType.DMA,
        ],
    )
    def gat(idx_h, z_h, out_h, i0, i1, rows0, rows1,
            si0, si1, sg0, sg1, so0, so1):
        core = lax.axis_index("c")
        sub = lax.axis_index("s")
        base = (sub * 2 + core) * per_w
        pltpu.async_copy(idx_h.at[pl.ds(base, EB)], i0, si0)
        pltpu.make_async_copy(idx_h.at[pl.ds(base, EB)], i0, si0).wait()
        pltpu.async_copy(z_h.at[i0], rows0, sg0)

        def pair(k, first):
            off0 = base + 2 * k * EB
            off1 = off0 + EB
            off2 = off0 + 2 * EB
            pltpu.async_copy(idx_h.at[pl.ds(off1, EB)], i1, si1)
            pltpu.make_async_copy(z_h.at[i0], rows0, sg0).wait()
            pltpu.async_copy(rows0, out_h.at[pl.ds(off0, EB)], so0)
            pltpu.make_async_copy(idx_h.at[pl.ds(off1, EB)], i1, si1).wait()
            if not first:
                pltpu.make_async_copy(rows1, out_h.at[pl.ds(off1, EB)],
                                      so1).wait()
            pltpu.async_copy(z_h.at[i1], rows1, sg1)
            pltpu.async_copy(idx_h.at[pl.ds(off2, EB)], i0, si0)
            pltpu.make_async_copy(z_h.at[i1], rows1, sg1).wait()
            pltpu.async_copy(rows1, out_h.at[pl.ds(off1, EB)], so1)
            pltpu.make_async_copy(idx_h.at[pl.ds(off2, EB)], i0, si0).wait()
            pltpu.make_async_copy(rows0, out_h.at[pl.ds(off0, EB)], so0).wait()
            pltpu.async_copy(z_h.at[i0], rows0, sg0)

        pair(0, True)
        lax.fori_loop(1, npairs, lambda k, c: (pair(k, False), c)[1], 0)
        pltpu.make_async_copy(z_h.at[i0], rows0, sg0).wait()
        pltpu.make_async_copy(rows1, out_h.at[pl.ds(base, EB)], so1).wait()

    return gat


@functools.lru_cache(maxsize=None)
def _mm_call(k):
    def body(x_ref, w_ref, b_ref, *o_refs):
        x = x_ref[...]
        for t in range(k):
            o_refs[t][...] = (
                jnp.dot(x, w_ref[:, t * 128:(t + 1) * 128],
                        preferred_element_type=jnp.float32)
                + b_ref[0, t * 128:(t + 1) * 128][None, :]
            )

    return pl.pallas_call(
        body,
        grid=(NP_ // BM,),
        in_specs=[
            pl.BlockSpec((BM, 128), lambda i: (i, 0)),
            pl.BlockSpec((128, 128 * k), lambda i: (0, 0)),
            pl.BlockSpec((8, 128 * k), lambda i: (0, 0)),
        ],
        out_specs=[pl.BlockSpec((BM, 128), lambda i: (i, 0))] * k,
        out_shape=[jax.ShapeDtypeStruct((NP_, 128), jnp.float32)] * k,
    )


@functools.lru_cache(maxsize=None)
def _comb_call(nt, relu):
    def body(*refs):
        o_ref = refs[-1]
        acc = None
        for t in range(nt):
            a = refs[2 * t][...] + refs[2 * t + 1][...]
            n2 = jnp.sum(a * a, axis=1, keepdims=True)
            a = a / jnp.maximum(jnp.sqrt(n2), 1e-12)
            acc = a if acc is None else acc + a
        if relu:
            acc = jnp.maximum(acc, 0.0)
        o_ref[...] = acc

    return pl.pallas_call(
        body,
        grid=(NP_ // BM,),
        in_specs=[pl.BlockSpec((BM, 128), lambda i: (i, 0))] * (2 * nt),
        out_specs=pl.BlockSpec((BM, 128), lambda i: (i, 0)),
        out_shape=jax.ShapeDtypeStruct((NP_, 128), jnp.float32),
    )


@functools.lru_cache(maxsize=None)
def _dec_call(blk_a, blk_b):
    DB = 8 * BM

    def body(za_ref, zb_ref, r_ref, o_ref):
        t = jnp.dot(za_ref[...], r_ref[...],
                    preferred_element_type=jnp.float32) * zb_ref[...]
        o_ref[...] = jnp.sum(t, axis=1).reshape(8, BM)

    return pl.pallas_call(
        body,
        grid=(GPAD // DB,),
        in_specs=[
            pl.BlockSpec((DB, 128), lambda i, o=blk_a: (i + o, 0)),
            pl.BlockSpec((DB, 128), lambda i, o=blk_b: (i + o, 0)),
            pl.BlockSpec((128, 128), lambda i: (0, 0)),
        ],
        out_specs=pl.BlockSpec((8, BM), lambda i: (i, 0)),
        out_shape=jax.ShapeDtypeStruct((GB, BM), jnp.float32),
    )


def _pad_rows(x):
    return jnp.concatenate(
        [x, jnp.zeros((NP_ - x.shape[0], x.shape[1]), x.dtype)], axis=0)


def _pad_edges(ei, epad):
    e = ei.shape[1]
    src = jnp.concatenate(
        [ei[0].astype(jnp.int32), jnp.zeros((epad - e,), jnp.int32)])
    dst = jnp.concatenate(
        [ei[1].astype(jnp.int32), jnp.full((epad - e,), ND, jnp.int32)])
    return jnp.stack(
        [src.reshape(-1, EB), dst.reshape(-1, EB)], axis=1).reshape(-1)


def _pad_idx(ix, extra=0):
    return jnp.concatenate(
        [ix.astype(jnp.int32),
         jnp.zeros((GPAD + extra - ix.shape[0],), jnp.int32)])


def kernel(x_drug, x_gene, ei_gene_interact_gene, ei_drug_has_target_gene, ei_gene_get_target_drug, ei_drug_rel0_drug, ei_drug_rel1_drug, Wm_gene_interact_gene_0, bm_gene_interact_gene_0, Ws_gene_interact_gene_0, bs_gene_interact_gene_0, Wm_drug_has_target_gene_0, bm_drug_has_target_gene_0, Ws_drug_has_target_gene_0, bs_drug_has_target_gene_0, Wm_gene_get_target_drug_0, bm_gene_get_target_drug_0, Ws_gene_get_target_drug_0, bs_gene_get_target_drug_0, Wm_drug_rel0_drug_0, bm_drug_rel0_drug_0, Ws_drug_rel0_drug_0, bs_drug_rel0_drug_0, Wm_drug_rel1_drug_0, bm_drug_rel1_drug_0, Ws_drug_rel1_drug_0, bs_drug_rel1_drug_0, Wm_gene_interact_gene_1, bm_gene_interact_gene_1, Ws_gene_interact_gene_1, bs_gene_interact_gene_1, Wm_drug_has_target_gene_1, bm_drug_has_target_gene_1, Ws_drug_has_target_gene_1, bs_drug_has_target_gene_1, Wm_gene_get_target_drug_1, bm_gene_get_target_drug_1, Ws_gene_get_target_drug_1, bs_gene_get_target_drug_1, Wm_drug_rel0_drug_1, bm_drug_rel0_drug_1, Ws_drug_rel0_drug_1, bs_drug_rel0_drug_1, Wm_drug_rel1_drug_1, bm_drug_rel1_drug_1, Ws_drug_rel1_drug_1, bs_drug_rel1_drug_1, R_bilinear_rel0, R_dedicom, D_dedicom_rel1):
    Wl = [
        (Wm_gene_interact_gene_0, bm_gene_interact_gene_0,
         Ws_gene_interact_gene_0, bs_gene_interact_gene_0,
         Wm_drug_has_target_gene_0, bm_drug_has_target_gene_0,
         Ws_drug_has_target_gene_0, bs_drug_has_target_gene_0,
         Wm_gene_get_target_drug_0, bm_gene_get_target_drug_0,
         Ws_gene_get_target_drug_0, bs_gene_get_target_drug_0,
         Wm_drug_rel0_drug_0, bm_drug_rel0_drug_0,
         Ws_drug_rel0_drug_0, bs_drug_rel0_drug_0,
         Wm_drug_rel1_drug_0, bm_drug_rel1_drug_0,
         Ws_drug_rel1_drug_0, bs_drug_rel1_drug_0),
        (Wm_gene_interact_gene_1, bm_gene_interact_gene_1,
         Ws_gene_interact_gene_1, bs_gene_interact_gene_1,
         Wm_drug_has_target_gene_1, bm_drug_has_target_gene_1,
         Ws_drug_has_target_gene_1, bs_drug_has_target_gene_1,
         Wm_gene_get_target_drug_1, bm_gene_get_target_drug_1,
         Ws_gene_get_target_drug_1, bs_gene_get_target_drug_1,
         Wm_drug_rel0_drug_1, bm_drug_rel0_drug_1,
         Ws_drug_rel0_drug_1, bs_drug_rel0_drug_1,
         Wm_drug_rel1_drug_1, bm_drug_rel1_drug_1,
         Ws_drug_rel1_drug_1, bs_drug_rel1_drug_1),
    ]
    sd_gg = _pad_edges(ei_gene_interact_gene, 200704)
    sd_dg = _pad_edges(ei_drug_has_target_gene, 100352)
    sd_gd = _pad_edges(ei_gene_get_target_drug, 100352)
    sd_d0 = _pad_edges(ei_drug_rel0_drug, 100352)
    sd_d1 = _pad_edges(ei_drug_rel1_drug, 100352)

    xg = _pad_rows(x_gene)
    xd = _pad_rows(x_drug)
    for l in range(2):
        (Wm_gg, bm_gg, Ws_gg, bs_gg,
         Wm_dg, bm_dg, Ws_dg, bs_dg,
         Wm_gd, bm_gd, Ws_gd, bs_gd,
         Wm_d0, bm_d0, Ws_d0, bs_d0,
         Wm_d1, bm_d1, Ws_d1, bs_d1) = Wl[l]
        Wg = jnp.concatenate([Wm_gg, Wm_gd, Ws_gg, Ws_dg], axis=1)
        bg = jnp.tile(jnp.concatenate([bm_gg, bm_gd, bs_gg, bs_dg])[None, :],
                      (8, 1))
        Wd = jnp.concatenate([Wm_dg, Wm_d0, Wm_d1, Ws_gd, Ws_d0, Ws_d1],
                             axis=1)
        bd = jnp.tile(
            jnp.concatenate([bm_dg, bm_d0, bm_d1, bs_gd, bs_d0, bs_d1])[None, :],
            (8, 1))
        Ymgg, Ymgd, Sgg, Sdg = _mm_call(4)(xg, Wg, bg)
        Ymdg, Ymd0, Ymd1, Sgd, Sd0, Sd1 = _mm_call(6)(xd, Wd, bd)
        agg_gg = _agg_call(200704)(sd_gg, Ymgg)
        agg_dg = _agg_call(100352)(sd_dg, Ymdg)
        agg_gd = _agg_call(100352)(sd_gd, Ymgd)
        agg_d0 = _agg_call(100352)(sd_d0, Ymd0)
        agg_d1 = _agg_call(100352)(sd_d1, Ymd1)
        relu = l == 0
        xg = _comb_call(2, relu)(agg_gg, Sgg, agg_dg, Sdg)
        xd = _comb_call(3, relu)(agg_gd, Sgd, agg_d0, Sd0, agg_d1, Sd1)

    idx_cat = jnp.concatenate([
        _pad_idx(ei_drug_rel0_drug[0]), _pad_idx(ei_drug_rel0_drug[1]),
        _pad_idx(ei_drug_rel1_drug[0]),
        _pad_idx(ei_drug_rel1_drug[1], extra=EB)])
    Zr = _gather_call(4 * GPAD)(idx_cat, xd)
    R1 = (D_dedicom_rel1[:, None] * R_dedicom) * D_dedicom_rel1[None, :]
    s0 = _dec_call(0, 49)(Zr, Zr, R_bilinear_rel0)
    s1 = _dec_call(98, 147)(Zr, Zr, R1)
    return jnp.concatenate(
        [s0.reshape(-1)[:100000], s1.reshape(-1)[:100000]])


# speedup vs baseline: 2.4018x; 1.0108x over previous
import functools

import jax
import jax.numpy as jnp
from jax import lax
from jax.experimental import pallas as pl
from jax.experimental.pallas import tpu as pltpu
from jax.experimental.pallas import tpu_sc as plsc

ND = 50000
NP_ = 50176
CHUNK = 12544
CT = CHUNK + 8
TPB = CHUNK // 16
EB = 128
EBA = 96
BM = 256
GPAD = 100352
GB = GPAD // BM


@functools.lru_cache(maxsize=None)
def _agg_call(epad):
    epw = epad // 16
    nsteps = epw // EB
    mesh = plsc.VectorSubcoreMesh(core_axis_name="c", subcore_axis_name="s")

    @functools.partial(
        pl.kernel,
        mesh=mesh,
        out_type=jax.ShapeDtypeStruct((NP_, 128), jnp.float32),
        scratch_types=[
            pltpu.VMEM((2 * EB,), jnp.int32),
            pltpu.VMEM((EB,), jnp.int32),
            pltpu.VMEM((EB, 128), jnp.float32),
            pltpu.VMEM((48, 128), jnp.float32),
            pltpu.VMEM_SHARED((CT, 128), jnp.float32),
            pltpu.SemaphoreType.DMA,
        ],
    )
    def agg(sd_h, y_h, out_h, sd_v, dl_v, rows_v, zb_v, agg_s, sem):
        core = lax.axis_index("c")
        sub = lax.axis_index("s")
        ebase = sub * epw * 2
        zero16 = jnp.zeros((16,), jnp.float32)

        def zrow(i, c):
            for j in range(8):
                zb_v[i, pl.ds(j * 16, 16)] = zero16
            return c

        lax.fori_loop(0, 48, zrow, 0)
        for p in range(2):
            lo = (p * 2 + core) * CHUNK
            for t in range(16):
                pltpu.sync_copy(zb_v, agg_s.at[pl.ds(sub * TPB + t * 48, 48)])
            pltpu.sync_copy(zb_v.at[pl.ds(0, 16)],
                            agg_s.at[pl.ds(sub * TPB + 768, 16)])
            plsc.subcore_barrier()

            def step(i, carry):
                off = ebase + i * (2 * EB)
                pltpu.sync_copy(sd_h.at[pl.ds(off, 2 * EB)], sd_v)
                pltpu.async_copy(y_h.at[sd_v.at[pl.ds(0, EB)]], rows_v,
                                 sem).wait()
                for j in range(8):
                    d16 = sd_v[pl.ds(EB + j * 16, 16)]
                    m = (d16 >= lo) & (d16 < lo + CHUNK)
                    dl_v[pl.ds(j * 16, 16)] = jnp.where(m, d16 - lo, CHUNK)
                pltpu.sync_copy(rows_v, agg_s.at[dl_v], add=True)
                return carry

            lax.fori_loop(0, nsteps, step, 0)
            plsc.subcore_barrier()
            pltpu.sync_copy(
                agg_s.at[pl.ds(sub * TPB, TPB)],
                out_h.at[pl.ds(lo + sub * TPB, TPB)],
            )
            plsc.subcore_barrier()

    return agg


@functools.lru_cache(maxsize=None)
def _gather_call(n):
    per_w = n // 32
    nsteps = per_w // EB
    mesh = plsc.VectorSubcoreMesh(core_axis_name="c", subcore_axis_name="s")

    npairs = nsteps // 2
    assert nsteps % 2 == 0

    @functools.partial(
        pl.kernel,
        mesh=mesh,
        out_type=jax.ShapeDtypeStruct((n, 128), jnp.float32),
        scratch_types=[
            pltpu.VMEM((EB,), jnp.int32), pltpu.VMEM((EB,), jnp.int32),
            pltpu.VMEM((EB, 128), jnp.float32),
            pltpu.VMEM((EB, 128), jnp.float32),
            pltpu.SemaphoreType.DMA, pltpu.SemaphoreType.DMA,
            pltpu.SemaphoreType.DMA, pltpu.SemaphoreType.DMA,
            pltpu.SemaphoreType.DMA, pltpu.SemaphoreType.DMA,
        ],
    )
    def gat(idx_h, z_h, out_h, i0, i1, rows0, rows1,
            si0, si1, sg0, sg1, so0, so1):
        core = lax.axis_index("c")
        sub = lax.axis_index("s")
        base = (sub * 2 + core) * per_w
        pltpu.async_copy(idx_h.at[pl.ds(base, EB)], i0, si0)
        pltpu.make_async_copy(idx_h.at[pl.ds(base, EB)], i0, si0).wait()
        pltpu.async_copy(z_h.at[i0], rows0, sg0)

        def pair(k, first):
            off0 = base + 2 * k * EB
            off1 = off0 + EB
            off2 = off0 + 2 * EB
            pltpu.async_copy(idx_h.at[pl.ds(off1, EB)], i1, si1)
            pltpu.make_async_copy(z_h.at[i0], rows0, sg0).wait()
            pltpu.async_copy(rows0, out_h.at[pl.ds(off0, EB)], so0)
            pltpu.make_async_copy(idx_h.at[pl.ds(off1, EB)], i1, si1).wait()
            if not first:
                pltpu.make_async_copy(rows1, out_h.at[pl.ds(off1, EB)],
                                      so1).wait()
            pltpu.async_copy(z_h.at[i1], rows1, sg1)
            pltpu.async_copy(idx_h.at[pl.ds(off2, EB)], i0, si0)
            pltpu.make_async_copy(z_h.at[i1], rows1, sg1).wait()
            pltpu.async_copy(rows1, out_h.at[pl.ds(off1, EB)], so1)
            pltpu.make_async_copy(idx_h.at[pl.ds(off2, EB)], i0, si0).wait()
            pltpu.make_async_copy(rows0, out_h.at[pl.ds(off0, EB)], so0).wait()
            pltpu.async_copy(z_h.at[i0], rows0, sg0)

        pair(0, True)
        lax.fori_loop(1, npairs, lambda k, c: (pair(k, False), c)[1], 0)
        pltpu.make_async_copy(z_h.at[i0], rows0, sg0).wait()
        pltpu.make_async_copy(rows1, out_h.at[pl.ds(base, EB)], so1).wait()

    return gat


@functools.lru_cache(maxsize=None)
def _mm_call(k):
    def body(x_ref, w_ref, b_ref, *o_refs):
        x = x_ref[...]
        for t in range(k):
            o_refs[t][...] = (
                jnp.dot(x, w_ref[:, t * 128:(t + 1) * 128],
                        preferred_element_type=jnp.float32)
                + b_ref[0, t * 128:(t + 1) * 128][None, :]
            )

    return pl.pallas_call(
        body,
        grid=(NP_ // BM,),
        in_specs=[
            pl.BlockSpec((BM, 128), lambda i: (i, 0)),
            pl.BlockSpec((128, 128 * k), lambda i: (0, 0)),
            pl.BlockSpec((8, 128 * k), lambda i: (0, 0)),
        ],
        out_specs=[pl.BlockSpec((BM, 128), lambda i: (i, 0))] * k,
        out_shape=[jax.ShapeDtypeStruct((NP_, 128), jnp.float32)] * k,
    )


@functools.lru_cache(maxsize=None)
def _comb_call(nt, relu):
    def body(*refs):
        o_ref = refs[-1]
        acc = None
        for t in range(nt):
            a = refs[2 * t][...] + refs[2 * t + 1][...]
            n2 = jnp.sum(a * a, axis=1, keepdims=True)
            a = a / jnp.maximum(jnp.sqrt(n2), 1e-12)
            acc = a if acc is None else acc + a
        if relu:
            acc = jnp.maximum(acc, 0.0)
        o_ref[...] = acc

    return pl.pallas_call(
        body,
        grid=(NP_ // BM,),
        in_specs=[pl.BlockSpec((BM, 128), lambda i: (i, 0))] * (2 * nt),
        out_specs=pl.BlockSpec((BM, 128), lambda i: (i, 0)),
        out_shape=jax.ShapeDtypeStruct((NP_, 128), jnp.float32),
    )


@functools.lru_cache(maxsize=None)
def _dec_call(blk_a, blk_b):
    DB = 8 * BM

    def body(za_ref, zb_ref, r_ref, o_ref):
        t = jnp.dot(za_ref[...], r_ref[...],
                    preferred_element_type=jnp.float32) * zb_ref[...]
        o_ref[...] = jnp.sum(t, axis=1).reshape(8, BM)

    return pl.pallas_call(
        body,
        grid=(GPAD // DB,),
        in_specs=[
            pl.BlockSpec((DB, 128), lambda i, o=blk_a: (i + o, 0)),
            pl.BlockSpec((DB, 128), lambda i, o=blk_b: (i + o, 0)),
            pl.BlockSpec((128, 128), lambda i: (0, 0)),
        ],
        out_specs=pl.BlockSpec((8, BM), lambda i: (i, 0)),
        out_shape=jax.ShapeDtypeStruct((GB, BM), jnp.float32),
    )


def _pad_rows(x):
    return jnp.concatenate(
        [x, jnp.zeros((NP_ - x.shape[0], x.shape[1]), x.dtype)], axis=0)


def _pad_edges(ei, epad):
    e = ei.shape[1]
    src = jnp.concatenate(
        [ei[0].astype(jnp.int32), jnp.zeros((epad - e,), jnp.int32)])
    dst = jnp.concatenate(
        [ei[1].astype(jnp.int32), jnp.full((epad - e,), ND, jnp.int32)])
    return jnp.stack(
        [src.reshape(-1, EB), dst.reshape(-1, EB)], axis=1).reshape(-1)


def _pad_idx(ix, extra=0):
    return jnp.concatenate(
        [ix.astype(jnp.int32),
         jnp.zeros((GPAD + extra - ix.shape[0],), jnp.int32)])


def kernel(x_drug, x_gene, ei_gene_interact_gene, ei_drug_has_target_gene, ei_gene_get_target_drug, ei_drug_rel0_drug, ei_drug_rel1_drug, Wm_gene_interact_gene_0, bm_gene_interact_gene_0, Ws_gene_interact_gene_0, bs_gene_interact_gene_0, Wm_drug_has_target_gene_0, bm_drug_has_target_gene_0, Ws_drug_has_target_gene_0, bs_drug_has_target_gene_0, Wm_gene_get_target_drug_0, bm_gene_get_target_drug_0, Ws_gene_get_target_drug_0, bs_gene_get_target_drug_0, Wm_drug_rel0_drug_0, bm_drug_rel0_drug_0, Ws_drug_rel0_drug_0, bs_drug_rel0_drug_0, Wm_drug_rel1_drug_0, bm_drug_rel1_drug_0, Ws_drug_rel1_drug_0, bs_drug_rel1_drug_0, Wm_gene_interact_gene_1, bm_gene_interact_gene_1, Ws_gene_interact_gene_1, bs_gene_interact_gene_1, Wm_drug_has_target_gene_1, bm_drug_has_target_gene_1, Ws_drug_has_target_gene_1, bs_drug_has_target_gene_1, Wm_gene_get_target_drug_1, bm_gene_get_target_drug_1, Ws_gene_get_target_drug_1, bs_gene_get_target_drug_1, Wm_drug_rel0_drug_1, bm_drug_rel0_drug_1, Ws_drug_rel0_drug_1, bs_drug_rel0_drug_1, Wm_drug_rel1_drug_1, bm_drug_rel1_drug_1, Ws_drug_rel1_drug_1, bs_drug_rel1_drug_1, R_bilinear_rel0, R_dedicom, D_dedicom_rel1):
    Wl = [
        (Wm_gene_interact_gene_0, bm_gene_interact_gene_0,
         Ws_gene_interact_gene_0, bs_gene_interact_gene_0,
         Wm_drug_has_target_gene_0, bm_drug_has_target_gene_0,
         Ws_drug_has_target_gene_0, bs_drug_has_target_gene_0,
         Wm_gene_get_target_drug_0, bm_gene_get_target_drug_0,
         Ws_gene_get_target_drug_0, bs_gene_get_target_drug_0,
         Wm_drug_rel0_drug_0, bm_drug_rel0_drug_0,
         Ws_drug_rel0_drug_0, bs_drug_rel0_drug_0,
         Wm_drug_rel1_drug_0, bm_drug_rel1_drug_0,
         Ws_drug_rel1_drug_0, bs_drug_rel1_drug_0),
        (Wm_gene_interact_gene_1, bm_gene_interact_gene_1,
         Ws_gene_interact_gene_1, bs_gene_interact_gene_1,
         Wm_drug_has_target_gene_1, bm_drug_has_target_gene_1,
         Ws_drug_has_target_gene_1, bs_drug_has_target_gene_1,
         Wm_gene_get_target_drug_1, bm_gene_get_target_drug_1,
         Ws_gene_get_target_drug_1, bs_gene_get_target_drug_1,
         Wm_drug_rel0_drug_1, bm_drug_rel0_drug_1,
         Ws_drug_rel0_drug_1, bs_drug_rel0_drug_1,
         Wm_drug_rel1_drug_1, bm_drug_rel1_drug_1,
         Ws_drug_rel1_drug_1, bs_drug_rel1_drug_1),
    ]
    sd_gg = _pad_edges(ei_gene_interact_gene, 200704)
    sd_dg = _pad_edges(ei_drug_has_target_gene, 100352)
    sd_gd = _pad_edges(ei_gene_get_target_drug, 100352)
    sd_d0 = _pad_edges(ei_drug_rel0_drug, 100352)
    sd_d1 = _pad_edges(ei_drug_rel1_drug, 100352)

    xg = _pad_rows(x_gene)
    xd = _pad_rows(x_drug)
    for l in range(2):
        (Wm_gg, bm_gg, Ws_gg, bs_gg,
         Wm_dg, bm_dg, Ws_dg, bs_dg,
         Wm_gd, bm_gd, Ws_gd, bs_gd,
         Wm_d0, bm_d0, Ws_d0, bs_d0,
         Wm_d1, bm_d1, Ws_d1, bs_d1) = Wl[l]
        Wg = jnp.concatenate([Wm_gg, Wm_gd, Ws_gg, Ws_dg], axis=1)
        bg = jnp.tile(jnp.concatenate([bm_gg, bm_gd, bs_gg, bs_dg])[None, :],
                      (8, 1))
        Wd = jnp.concatenate([Wm_dg, Wm_d0, Wm_d1, Ws_gd, Ws_d0, Ws_d1],
                             axis=1)
        bd = jnp.tile(
            jnp.concatenate([bm_dg, bm_d0, bm_d1, bs_gd, bs_d0, bs_d1])[None, :],
            (8, 1))
        Ymgg, Ymgd, Sgg, Sdg = _mm_call(4)(xg, Wg, bg)
        Ymdg, Ymd0, Ymd1, Sgd, Sd0, Sd1 = _mm_call(6)(xd, Wd, bd)
        agg_gg = _agg_call(200704)(sd_gg, Ymgg)
        agg_dg = _agg_call(100352)(sd_dg, Ymdg)
        agg_gd = _agg_call(100352)(sd_gd, Ymgd)
        agg_d0 = _agg_call(100352)(sd_d0, Ymd0)
        agg_d1 = _agg_call(100352)(sd_d1, Ymd1)
        relu = l == 0
        xg = _comb_call(2, relu)(agg_gg, Sgg, agg_dg, Sdg)
        xd = _comb_call(3, relu)(agg_gd, Sgd, agg_d0, Sd0, agg_d1, Sd1)

    idx_cat = jnp.concatenate([
        _pad_idx(ei_drug_rel0_drug[0]), _pad_idx(ei_drug_rel0_drug[1]),
        _pad_idx(ei_drug_rel1_drug[0]),
        _pad_idx(ei_drug_rel1_drug[1], extra=EB)])
    Zr = _gather_call(4 * GPAD)(idx_cat, xd)
    R1 = (D_dedicom_rel1[:, None] * R_dedicom) * D_dedicom_rel1[None, :]
    s0 = _dec_call(0, 49)(Zr, Zr, R_bilinear_rel0)
    s1 = _dec_call(98, 147)(Zr, Zr, R1)
    return jnp.concatenate(
        [s0.reshape(-1)[:100000], s1.reshape(-1)[:100000]])
